# Initial kernel scaffold; baseline (speedup 1.0000x reference)
import jax
import jax.numpy as jnp
from jax.experimental import pallas as pl


def kernel(center, contexts, neg_samples, ivec_w, ovec_w):
    raise NotImplementedError("write your pallas kernel here")


# speedup vs baseline: 4.7851x; 4.7851x over previous
import functools

import jax
import jax.numpy as jnp
from jax import lax
from jax.experimental import pallas as pl
from jax.experimental.pallas import tpu as pltpu
from jax.experimental.pallas import tpu_sc as plsc

_VOCAB = 1000000
_D = 64
_B = 16384
_C = 20
_N = 20
_J = _C + _N
_JP = 48

_NC, _NS = 2, 16
_NW = _NC * _NS
_BPW = _B // _NW
_CH = 16
_NCHUNK = _BPW // _CH
_R = _CH * _J
_G = _R // 128


def _sc_scores_kernel(ivec_hbm, ovec_hbm, idx_hbm, cen_hbm, out_hbm,
                      idx_v, cen_v, iv_v, ov_v, sc_v, sem):
    wid = lax.axis_index("s") * _NC + lax.axis_index("c")
    lane = lax.iota(jnp.int32, 16)

    def chunk_body(ch, carry):
        blk = wid * _NCHUNK + ch
        pltpu.sync_copy(idx_hbm.at[pl.ds(blk * _R, _R)], idx_v)
        pltpu.sync_copy(cen_hbm.at[pl.ds(blk * _CH, _CH)], cen_v)
        pltpu.async_copy(ivec_hbm.at[cen_v], iv_v, sem).wait()
        for g in range(_G):
            pltpu.async_copy(ovec_hbm.at[idx_v.at[pl.ds(g * 128, 128)]],
                             ov_v.at[pl.ds(g * 128, 128)], sem).wait()

        perms = [(lane ^ step).astype(jnp.int32) for step in (1, 2, 4, 8)]

        def center_body(c, carry2):
            iv = [iv_v[c, pl.ds(k * 16, 16)] for k in range(4)]
            accs = [jnp.zeros((16,), jnp.float32) for _ in range(3)]
            for j in range(_J):
                r = c * _J + j
                p = ov_v[r, pl.ds(0, 16)] * iv[0]
                for k in range(1, 4):
                    p = p + ov_v[r, pl.ds(k * 16, 16)] * iv[k]
                for perm in perms:
                    p = p + p.at[perm].get(mode="promise_in_bounds")
                a = j // 16
                accs[a] = jnp.where(lane == (j % 16), p, accs[a])
            for a in range(3):
                sc_v[c, pl.ds(a * 16, 16)] = accs[a]
            return carry2

        lax.fori_loop(0, _CH, center_body, 0, unroll=False)
        pltpu.sync_copy(sc_v, out_hbm.at[pl.ds(blk * _CH, _CH)])
        return carry

    lax.fori_loop(0, _NCHUNK, chunk_body, 0, unroll=False)


def _sc_scores(ivec_w, ovec_w, idx2d, cen2d):
    mesh = plsc.VectorSubcoreMesh(core_axis_name="c", subcore_axis_name="s")
    return pl.kernel(
        _sc_scores_kernel,
        out_type=jax.ShapeDtypeStruct((_B, _JP), jnp.float32),
        mesh=mesh,
        scratch_types=[
            pltpu.VMEM((_R,), jnp.int32),
            pltpu.VMEM((_CH,), jnp.int32),
            pltpu.VMEM((_CH, _D), jnp.float32),
            pltpu.VMEM((_R, _D), jnp.float32),
            pltpu.VMEM((_CH, _JP), jnp.float32),
            pltpu.SemaphoreType.DMA,
        ],
        compiler_params=pltpu.CompilerParams(use_tc_tiling_on_sc=False),
    )(ivec_w, ovec_w, idx2d, cen2d)


def _loss_body(s_ref, i_ref, o_ref):
    @pl.when(pl.program_id(0) == 0)
    def _():
        o_ref[0, 0] = 0.0

    m = (i_ref[...] != 0).astype(jnp.float32)
    sm = s_ref[:, : _J] * m
    ls_o = jnp.log(jax.nn.sigmoid(sm[:, : _C]))
    ls_n = jnp.log(jax.nn.sigmoid(-sm[:, _C:]))
    o_ref[0, 0] += -(jnp.sum(ls_o) / _C + jnp.sum(ls_n) / _N) / _B


def _loss_tc(scores, idx):
    blk = 2048
    out = pl.pallas_call(
        _loss_body,
        grid=(_B // blk,),
        in_specs=[
            pl.BlockSpec((blk, _JP), lambda i: (i, 0)),
            pl.BlockSpec((blk, _J), lambda i: (i, 0)),
        ],
        out_specs=pl.BlockSpec((1, 1), lambda i: (0, 0),
                               memory_space=pltpu.SMEM),
        out_shape=jax.ShapeDtypeStruct((1, 1), jnp.float32),
    )(scores, idx)
    return out[0, 0]


def kernel(center, contexts, neg_samples, ivec_w, ovec_w):
    center = center.astype(jnp.int32)
    idx = jnp.concatenate([contexts, neg_samples], axis=1).astype(jnp.int32)
    idx = jnp.where(center[:, None] != 0, idx, 0)
    scores = _sc_scores(ivec_w, ovec_w, idx.reshape(_B * _J), center)
    return _loss_tc(scores, idx)
